# baseline (speedup 1.0000x reference)
import functools

import jax
import jax.numpy as jnp
from jax import lax
from jax.experimental import pallas as pl
from jax.experimental.pallas import tpu as pltpu
from jax.experimental.pallas import tpu_sc as plsc


def _mask_update_sc(wm_flat, bm_flat):
    n = wm_flat.shape[0]
    info = plsc.get_sparse_core_info()
    nw = info.num_cores * info.num_subcores
    per = n // nw
    nseg = per // 16

    @functools.partial(
        pl.kernel,
        out_type=[jax.ShapeDtypeStruct((n,), jnp.float32),
                  jax.ShapeDtypeStruct((n,), jnp.float32)],
        scratch_types=[pltpu.VMEM((per,), jnp.float32),
                       pltpu.VMEM((per,), jnp.float32),
                       pltpu.VMEM((per,), jnp.float32),
                       pltpu.VMEM((per,), jnp.float32)],
        mesh=plsc.VectorSubcoreMesh(core_axis_name="c", subcore_axis_name="s"),
    )
    def k(wm_hbm, bm_hbm, nwm_hbm, nbm_hbm, wm_v, bm_v, nwm_v, nbm_v):
        wid = lax.axis_index("s") * info.num_cores + lax.axis_index("c")
        base = wid * per
        pltpu.sync_copy(wm_hbm.at[pl.ds(base, per)], wm_v)
        pltpu.sync_copy(bm_hbm.at[pl.ds(base, per)], bm_v)
        lane = lax.iota(jnp.int32, 16)
        roll_idx = (lane + 15) % 16
        for j in range(nseg):
            w = wm_v[pl.ds(j * 16, 16)]
            b = bm_v[pl.ds(j * 16, 16)]
            nwm_v[pl.ds(j * 16, 16)] = lax.gather(
                w, roll_idx[:, None],
                dimension_numbers=lax.GatherDimensionNumbers(
                    offset_dims=(), collapsed_slice_dims=(0,),
                    start_index_map=(0,)),
                slice_sizes=(1,),
                mode=lax.GatherScatterMode.PROMISE_IN_BOUNDS)
            nbm_v[pl.ds(j * 16, 16)] = jnp.minimum(w + b, 1.0)
        pltpu.sync_copy(nwm_v, nwm_hbm.at[pl.ds(base, per)])
        pltpu.sync_copy(nbm_v, nbm_hbm.at[pl.ds(base, per)])

    return k(wm_flat, bm_flat)


def _body(batch_ref, z_ref, mv2d_ref, mvm2d_ref, wm_ref, bm_ref,
          WvT_ref, bv_ref, Wa1T_ref, ba1_ref, Katt_ref, ba2t_ref,
          WhSum_ref, bh_ref, WoT_ref, bo_ref,
          S4_ref, GS_ref,
          out_ref, nmv_ref,
          acc_ref, att_ref, mvWo_ref,
          *, T, B, M, E, H, OUT, n_tiles):
    i = pl.program_id(0)
    f32 = jnp.float32
    bf16 = jnp.bfloat16

    @pl.when(i == 0)
    def _prologue():
        acc_ref[...] = jnp.zeros((B, E), dtype=f32)
        att2 = jnp.dot(mv2d_ref[...], Katt_ref[...],
                       preferred_element_type=f32) + ba2t_ref[...]
        biast = (bm_ref[...] - 1.0) * 1e9
        att_ref[...] = jnp.concatenate(
            [att2, biast, biast, biast, biast], axis=1)
        for m in range(M):
            blk = jnp.dot(mvm2d_ref[m * B:(m + 1) * B, :].astype(bf16),
                          WoT_ref[...], preferred_element_type=f32)
            mvWo_ref[m * B:(m + 1) * B, :] = blk.astype(bf16)

    b_t = batch_ref[0]
    oh = (b_t == jax.lax.broadcasted_iota(jnp.int32, (T, B), 1)).astype(bf16)

    z_t = z_ref[...].astype(bf16)
    wv = (jnp.dot(z_t, WvT_ref[...], preferred_element_type=f32)
          + bv_ref[...]).astype(bf16)
    acc_ref[...] += jax.lax.dot_general(
        oh, wv, (((0,), (0,)), ((), ())), preferred_element_type=f32)

    a1 = jnp.dot(z_t, Wa1T_ref[...], preferred_element_type=f32) + ba1_ref[...]
    a1t = jnp.dot(a1, S4_ref[...], preferred_element_type=f32)

    r_att = jnp.dot(oh, att_ref[...], preferred_element_type=f32)
    l = r_att[:, :H * M] + a1t
    l = jnp.where(l >= 0, l, 0.01 * l) + r_att[:, H * M:]
    e = jnp.exp(l)
    sums4 = jnp.dot(e, GS_ref[...], preferred_element_type=f32)
    c = e * jnp.dot(1.0 / sums4, S4_ref[...], preferred_element_type=f32)
    s = jnp.dot(c, WhSum_ref[...], preferred_element_type=f32) + bh_ref[0, 0]
    es = jnp.exp(s)
    coefs = (es / jnp.sum(es, axis=1, keepdims=True)).astype(bf16)

    fin = jnp.zeros((T, OUT), dtype=f32)
    for m in range(M):
        fin = fin + jnp.dot(coefs[:, m:m + 1] * oh,
                            mvWo_ref[m * B:(m + 1) * B, :],
                            preferred_element_type=f32)
    out_ref[...] = fin + bo_ref[...]

    @pl.when(i == n_tiles - 1)
    def _epilogue():
        wv_seg = jnp.tanh(acc_ref[...])
        wm = wm_ref[...]
        for m in range(M):
            nmv_ref[:, m * E:(m + 1) * E] = (
                mv2d_ref[:, m * E:(m + 1) * E] + wv_seg * wm[:, m:m + 1])


def kernel(z, batch, memory_values, write_mask, bias_mask,
           Wv, bv, Wa1, ba1, Wa2, ba2, Wh, bh, Wo, bo):
    N, F_IN = z.shape
    B, M, E = memory_values.shape
    H = Wa1.shape[0]
    OUT = Wo.shape[0]
    T = 5000
    n_tiles = -(-N // T)
    Npad = n_tiles * T
    f32 = jnp.float32
    bf16 = jnp.bfloat16

    batch_i = batch.astype(jnp.int32)
    batch3 = jnp.pad(batch_i, (0, Npad - N), constant_values=B).reshape(
        n_tiles, T, 1)
    z_p = jnp.pad(z, ((0, Npad - N), (0, 0)))

    mv2d = memory_values.reshape(B, M * E)
    mvm2d = jnp.transpose(memory_values, (1, 0, 2)).reshape(M * B, E)

    eyeM = jnp.eye(M, dtype=f32)
    Katt = jnp.concatenate(
        [jnp.kron(eyeM, Wa2[h].reshape(E, 1)) for h in range(H)], axis=1)
    ba2t = jnp.repeat(ba2, M).reshape(1, H * M).astype(f32)
    S4 = jnp.kron(jnp.eye(H, dtype=f32), jnp.ones((1, M), f32))
    GS = jnp.kron(jnp.eye(H, dtype=f32), jnp.ones((M, 1), f32))
    WhSum = jnp.kron(Wh.reshape(H, 1), eyeM)

    body = functools.partial(_body, T=T, B=B, M=M, E=E, H=H, OUT=OUT,
                             n_tiles=n_tiles)

    full = lambda shape: pl.BlockSpec(shape, lambda i: (0,) * len(shape))
    nwm_flat, nbm_flat = _mask_update_sc(
        write_mask.reshape(B * M), bias_mask.reshape(B * M))

    out, nmv2d = pl.pallas_call(
        body,
        grid=(n_tiles,),
        in_specs=[
            pl.BlockSpec((1, T, 1), lambda i: (i, 0, 0)),
            pl.BlockSpec((T, F_IN), lambda i: (i, 0)),
            full((B, M * E)),
            full((M * B, E)),
            full((B, M)),
            full((B, M)),
            full((F_IN, E)),
            full((1, E)),
            full((F_IN, H)),
            full((1, H)),
            full((M * E, H * M)),
            full((1, H * M)),
            full((H * M, M)),
            full((1, 1)),
            full((E, OUT)),
            full((1, OUT)),
            full((H, H * M)),
            full((H * M, H)),
        ],
        out_specs=[
            pl.BlockSpec((T, OUT), lambda i: (i, 0)),
            full((B, M * E)),
        ],
        out_shape=[
            jax.ShapeDtypeStruct((Npad, OUT), f32),
            jax.ShapeDtypeStruct((B, M * E), f32),
        ],
        scratch_shapes=[
            pltpu.VMEM((B, E), f32),
            pltpu.VMEM((B, 2 * H * M), f32),
            pltpu.VMEM((M * B, OUT), bf16),
        ],
    )(batch3, z_p, mv2d, mvm2d, write_mask, bias_mask,
      Wv.T.astype(bf16), bv.reshape(1, E), Wa1.T.astype(bf16),
      ba1.reshape(1, H), Katt, ba2t,
      WhSum, bh.reshape(1, 1), Wo.T.astype(bf16), bo.reshape(1, OUT),
      S4, GS)

    output = out[:N]
    new_memory_values = nmv2d.reshape(B, M, E)
    return (output, jnp.arange(N), new_memory_values,
            nwm_flat.reshape(B, M), nbm_flat.reshape(B, M))


# speedup vs baseline: 1.0818x; 1.0818x over previous
import functools

import jax
import jax.numpy as jnp
from jax import lax
from jax.experimental import pallas as pl
from jax.experimental.pallas import tpu as pltpu
from jax.experimental.pallas import tpu_sc as plsc


def _mask_update_sc(wm_flat, bm_flat):
    n = wm_flat.shape[0]
    info = plsc.get_sparse_core_info()
    nw = info.num_cores * info.num_subcores
    per = n // nw
    nseg = per // 16

    @functools.partial(
        pl.kernel,
        out_type=[jax.ShapeDtypeStruct((n,), jnp.float32),
                  jax.ShapeDtypeStruct((n,), jnp.float32)],
        scratch_types=[pltpu.VMEM((per,), jnp.float32),
                       pltpu.VMEM((per,), jnp.float32),
                       pltpu.VMEM((per,), jnp.float32),
                       pltpu.VMEM((per,), jnp.float32)],
        mesh=plsc.VectorSubcoreMesh(core_axis_name="c", subcore_axis_name="s"),
    )
    def k(wm_hbm, bm_hbm, nwm_hbm, nbm_hbm, wm_v, bm_v, nwm_v, nbm_v):
        wid = lax.axis_index("s") * info.num_cores + lax.axis_index("c")
        base = wid * per
        pltpu.sync_copy(wm_hbm.at[pl.ds(base, per)], wm_v)
        pltpu.sync_copy(bm_hbm.at[pl.ds(base, per)], bm_v)
        lane = lax.iota(jnp.int32, 16)
        roll_idx = (lane + 15) % 16
        for j in range(nseg):
            w = wm_v[pl.ds(j * 16, 16)]
            b = bm_v[pl.ds(j * 16, 16)]
            nwm_v[pl.ds(j * 16, 16)] = lax.gather(
                w, roll_idx[:, None],
                dimension_numbers=lax.GatherDimensionNumbers(
                    offset_dims=(), collapsed_slice_dims=(0,),
                    start_index_map=(0,)),
                slice_sizes=(1,),
                mode=lax.GatherScatterMode.PROMISE_IN_BOUNDS)
            nbm_v[pl.ds(j * 16, 16)] = jnp.minimum(w + b, 1.0)
        pltpu.sync_copy(nwm_v, nwm_hbm.at[pl.ds(base, per)])
        pltpu.sync_copy(nbm_v, nbm_hbm.at[pl.ds(base, per)])

    return k(wm_flat, bm_flat)


def _body(batch_ref, z_ref, mv2d_ref, mvm2d_ref, wm_ref, bm_ref,
          WvT_ref, bv_ref, Wa1T_ref, ba1_ref, Katt_ref, ba2t_ref,
          WhSum_ref, bh_ref, WoT_ref, bo_ref,
          S4_ref, GS_ref,
          out_ref, nmv_ref,
          acc_ref, att_ref, mvWo_ref,
          *, T, B, M, E, H, OUT, n_tiles):
    i = pl.program_id(0)
    f32 = jnp.float32
    bf16 = jnp.bfloat16

    @pl.when(i == 0)
    def _prologue():
        acc_ref[...] = jnp.zeros((B, E), dtype=f32)
        att2 = jnp.dot(mv2d_ref[...], Katt_ref[...],
                       preferred_element_type=f32) + ba2t_ref[...]
        biast = (bm_ref[...] - 1.0) * 1e9
        att_ref[...] = jnp.concatenate(
            [att2, biast, biast, biast, biast], axis=1)
        for m in range(M):
            blk = jnp.dot(mvm2d_ref[m * B:(m + 1) * B, :].astype(bf16),
                          WoT_ref[...], preferred_element_type=f32)
            mvWo_ref[m * B:(m + 1) * B, :] = blk.astype(bf16)

    b_t = batch_ref[0]
    oh = (b_t == jax.lax.broadcasted_iota(jnp.int32, (T, B), 1)).astype(bf16)

    z_t = z_ref[...].astype(bf16)
    wv = (jnp.dot(z_t, WvT_ref[...], preferred_element_type=f32)
          + bv_ref[...]).astype(bf16)
    acc_ref[...] += jax.lax.dot_general(
        oh, wv, (((0,), (0,)), ((), ())), preferred_element_type=f32)

    a1 = jnp.dot(z_t, Wa1T_ref[...], preferred_element_type=f32) + ba1_ref[...]
    a1t = jnp.dot(a1, S4_ref[...], preferred_element_type=f32)

    r_att = jnp.dot(oh, att_ref[...], preferred_element_type=f32)
    l = r_att[:, :H * M] + a1t
    l = jnp.where(l >= 0, l, 0.01 * l) + r_att[:, H * M:]
    e = jnp.exp(l)
    sums4 = jnp.dot(e, GS_ref[...], preferred_element_type=f32)
    c = e * jnp.dot(1.0 / sums4, S4_ref[...], preferred_element_type=f32)
    s = jnp.dot(c, WhSum_ref[...], preferred_element_type=f32) + bh_ref[0, 0]
    es = jnp.exp(s)
    coefs = (es / jnp.sum(es, axis=1, keepdims=True)).astype(bf16)

    fin = jnp.zeros((T, OUT), dtype=f32)
    for m in range(M):
        fin = fin + jnp.dot(coefs[:, m:m + 1] * oh,
                            mvWo_ref[m * B:(m + 1) * B, :],
                            preferred_element_type=f32)
    out_ref[...] = fin + bo_ref[...]

    @pl.when(i == n_tiles - 1)
    def _epilogue():
        wv_seg = jnp.tanh(acc_ref[...])
        wm = wm_ref[...]
        for m in range(M):
            nmv_ref[:, m * E:(m + 1) * E] = (
                mv2d_ref[:, m * E:(m + 1) * E] + wv_seg * wm[:, m:m + 1])


def kernel(z, batch, memory_values, write_mask, bias_mask,
           Wv, bv, Wa1, ba1, Wa2, ba2, Wh, bh, Wo, bo):
    N, F_IN = z.shape
    B, M, E = memory_values.shape
    H = Wa1.shape[0]
    OUT = Wo.shape[0]
    T = 2000
    n_tiles = -(-N // T)
    Npad = n_tiles * T
    f32 = jnp.float32
    bf16 = jnp.bfloat16

    batch_i = batch.astype(jnp.int32)
    batch3 = jnp.pad(batch_i, (0, Npad - N), constant_values=B).reshape(
        n_tiles, T, 1)
    z_p = jnp.pad(z, ((0, Npad - N), (0, 0)))

    mv2d = memory_values.reshape(B, M * E)
    mvm2d = jnp.transpose(memory_values, (1, 0, 2)).reshape(M * B, E)

    eyeM = jnp.eye(M, dtype=f32)
    Katt = jnp.concatenate(
        [jnp.kron(eyeM, Wa2[h].reshape(E, 1)) for h in range(H)], axis=1)
    ba2t = jnp.repeat(ba2, M).reshape(1, H * M).astype(f32)
    S4 = jnp.kron(jnp.eye(H, dtype=f32), jnp.ones((1, M), f32))
    GS = jnp.kron(jnp.eye(H, dtype=f32), jnp.ones((M, 1), f32))
    WhSum = jnp.kron(Wh.reshape(H, 1), eyeM)

    body = functools.partial(_body, T=T, B=B, M=M, E=E, H=H, OUT=OUT,
                             n_tiles=n_tiles)

    full = lambda shape: pl.BlockSpec(shape, lambda i: (0,) * len(shape))
    nwm_flat, nbm_flat = _mask_update_sc(
        write_mask.reshape(B * M), bias_mask.reshape(B * M))

    out, nmv2d = pl.pallas_call(
        body,
        grid=(n_tiles,),
        in_specs=[
            pl.BlockSpec((1, T, 1), lambda i: (i, 0, 0)),
            pl.BlockSpec((T, F_IN), lambda i: (i, 0)),
            full((B, M * E)),
            full((M * B, E)),
            full((B, M)),
            full((B, M)),
            full((F_IN, E)),
            full((1, E)),
            full((F_IN, H)),
            full((1, H)),
            full((M * E, H * M)),
            full((1, H * M)),
            full((H * M, M)),
            full((1, 1)),
            full((E, OUT)),
            full((1, OUT)),
            full((H, H * M)),
            full((H * M, H)),
        ],
        out_specs=[
            pl.BlockSpec((T, OUT), lambda i: (i, 0)),
            full((B, M * E)),
        ],
        out_shape=[
            jax.ShapeDtypeStruct((Npad, OUT), f32),
            jax.ShapeDtypeStruct((B, M * E), f32),
        ],
        scratch_shapes=[
            pltpu.VMEM((B, E), f32),
            pltpu.VMEM((B, 2 * H * M), f32),
            pltpu.VMEM((M * B, OUT), bf16),
        ],
    )(batch3, z_p, mv2d, mvm2d, write_mask, bias_mask,
      Wv.T.astype(bf16), bv.reshape(1, E), Wa1.T.astype(bf16),
      ba1.reshape(1, H), Katt, ba2t,
      WhSum, bh.reshape(1, 1), Wo.T.astype(bf16), bo.reshape(1, OUT),
      S4, GS)

    output = out[:N]
    new_memory_values = nmv2d.reshape(B, M, E)
    return (output, jnp.arange(N), new_memory_values,
            nwm_flat.reshape(B, M), nbm_flat.reshape(B, M))
